# baseline (speedup 1.0000x reference)
import functools

import jax
import jax.numpy as jnp
from jax import lax
from jax.experimental import pallas as pl
from jax.experimental.pallas import tpu as pltpu
from jax.experimental.pallas import tpu_sc as plsc

BATCH = 16384
D = 64
NC, NS, L = 2, 16, 16
NW = NC * NS
BPW = BATCH // NW
CH = 128
NCH = BPW // CH
NBUF = 2


def _fast_sqrt(x):
    i = lax.bitcast_convert_type(x, jnp.int32)
    y = lax.bitcast_convert_type(
        jnp.int32(0x5F3759DF) - lax.shift_right_arithmetic(i, 1), jnp.float32)
    h = 0.5 * x
    y = y * (1.5 - (h * y) * y)
    y = y * (1.5 - (h * y) * y)
    y = y * (1.5 - (h * y) * y)
    return x * y


def _sc_body(u_hbm, v_hbm, re_hbm, im_hbm, out_hbm,
             uidx, vidx, gbufs, outb, sems):
    wid = lax.axis_index("s") * NC + lax.axis_index("c")
    base = wid * BPW

    def fire(c, slot):
        off = base + c * CH
        pltpu.sync_copy(u_hbm.at[pl.ds(off, CH)], uidx[slot])
        pltpu.sync_copy(v_hbm.at[pl.ds(off, CH)], vidx[slot])
        gru, giu, grv, giv = gbufs[slot]
        return [
            pltpu.async_copy(re_hbm.at[uidx[slot]], gru, sems[slot]),
            pltpu.async_copy(im_hbm.at[uidx[slot]], giu, sems[slot]),
            pltpu.async_copy(re_hbm.at[vidx[slot]], grv, sems[slot]),
            pltpu.async_copy(im_hbm.at[vidx[slot]], giv, sems[slot]),
        ]

    def compute(c, slot):
        gru, giu, grv, giv = gbufs[slot]

        def group(g, _):
            rows = lax.iota(jnp.int32, L) + g * L
            def dbody(d, acc):
                cold = jnp.full((L,), d, jnp.int32)
                ru = plsc.load_gather(gru, [rows, cold])
                iu = plsc.load_gather(giu, [rows, cold])
                rv = plsc.load_gather(grv, [rows, cold])
                iv = plsc.load_gather(giv, [rows, cold])
                p = (ru * ru + iu * iu) * (rv * rv + iv * iv)
                return acc + _fast_sqrt(p)
            acc = lax.fori_loop(0, D, dbody, jnp.zeros((L,), jnp.float32))
            e = jnp.exp(-acc)
            outb[pl.ds(c * CH + g * L, L)] = e / (1.0 + e)
            return 0

        lax.fori_loop(0, CH // L, group, 0)

    handles = [None] * NBUF
    handles[0] = fire(0, 0)
    for c in range(NCH):
        slot = c % NBUF
        nxt = c + 1
        if nxt < NCH:
            handles[nxt % NBUF] = fire(nxt, nxt % NBUF)
        for h in handles[slot]:
            h.wait()
        compute(c, slot)

    pltpu.sync_copy(outb, out_hbm.at[pl.ds(base, BPW)])


@jax.jit
def _rotate_score(u, v, emb_re, emb_im):
    mesh = plsc.VectorSubcoreMesh(core_axis_name="c", subcore_axis_name="s")
    gather_bufs = [
        [pltpu.VMEM((CH, D), jnp.float32) for _ in range(4)]
        for _ in range(NBUF)
    ]
    run = pl.kernel(
        _sc_body,
        out_type=jax.ShapeDtypeStruct((BATCH,), jnp.float32),
        mesh=mesh,
        scratch_types=dict(
            uidx=[pltpu.VMEM((CH,), jnp.int32) for _ in range(NBUF)],
            vidx=[pltpu.VMEM((CH,), jnp.int32) for _ in range(NBUF)],
            gbufs=gather_bufs,
            outb=pltpu.VMEM((BPW,), jnp.float32),
            sems=[pltpu.SemaphoreType.DMA for _ in range(NBUF)],
        ),
        compiler_params=pltpu.CompilerParams(needs_layout_passes=False,
                                             use_tc_tiling_on_sc=False),
    )
    return run(u, v, emb_re, emb_im)


def kernel(u, v, emb_re, emb_im):
    return _rotate_score(u.astype(jnp.int32), v.astype(jnp.int32),
                         emb_re, emb_im)


# speedup vs baseline: 3.3559x; 3.3559x over previous
import jax
import jax.numpy as jnp
from jax import lax
from jax.experimental import pallas as pl
from jax.experimental.pallas import tpu as pltpu
from jax.experimental.pallas import tpu_sc as plsc

N_ENT = 1000000
BATCH = 16384
D = 64
NC, NS, L = 2, 16, 16
NW = NC * NS
BPW = BATCH // NW
CH = 128
NCH = BPW // CH
NBUF = 2
NB = 4096
GRID1 = 123
SPLIT = (GRID1 - 1) * NB
OUT_ROWS = GRID1 * NB


def _mod_body(reL, imL, reR, imR, out):
    eye = jnp.eye(D, dtype=jnp.float32)
    dn = (((0,), (0,)), ((), ()))
    mL = jnp.sqrt(reL[...] * reL[...] + imL[...] * imL[...])
    mR = jnp.sqrt(reR[...] * reR[...] + imR[...] * imR[...])
    out[:, 0:D] = lax.dot_general(mL, eye, dn,
                                  preferred_element_type=jnp.float32)
    out[:, D:2 * D] = lax.dot_general(mR, eye, dn,
                                      preferred_element_type=jnp.float32)


def _mod_table(re_t, im_t):
    return pl.pallas_call(
        _mod_body,
        grid=(GRID1,),
        in_specs=[
            pl.BlockSpec((D, NB), lambda i: (0, i)),
            pl.BlockSpec((D, NB), lambda i: (0, i)),
            pl.BlockSpec((D, NB), lambda i: (0, i + GRID1 - 1)),
            pl.BlockSpec((D, NB), lambda i: (0, i + GRID1 - 1)),
        ],
        out_specs=pl.BlockSpec((NB, 2 * D), lambda i: (i, 0)),
        out_shape=jax.ShapeDtypeStruct((OUT_ROWS, 2 * D), jnp.float32),
    )(re_t, im_t, re_t, im_t)


def _score_body(u_hbm, v_hbm, mod_hbm, out_hbm,
                uraw, vraw, pu, pv, cu, cv, gu, gv, outb, sems):
    wid = lax.axis_index("s") * NC + lax.axis_index("c")
    base = wid * BPW

    def stage_idx(raw, p, c):
        for j in range(CH // L):
            r = raw[pl.ds(j * L, L)]
            hi = r >= SPLIT
            p[pl.ds(j * L, L)] = jnp.where(hi, r - SPLIT, r)
            c[pl.ds(j * L, L)] = jnp.where(hi, D, 0)

    def fire(ch, slot):
        off = base + ch * CH
        pltpu.sync_copy(u_hbm.at[pl.ds(off, CH)], uraw[slot])
        pltpu.sync_copy(v_hbm.at[pl.ds(off, CH)], vraw[slot])
        stage_idx(uraw[slot], pu[slot], cu[slot])
        stage_idx(vraw[slot], pv[slot], cv[slot])
        return [
            pltpu.async_copy(mod_hbm.at[pu[slot]], gu[slot], sems[slot]),
            pltpu.async_copy(mod_hbm.at[pv[slot]], gv[slot], sems[slot]),
        ]

    def compute(ch, slot):
        def group(g, _):
            rows = lax.iota(jnp.int32, L) + g * L
            cub = cu[slot][pl.ds(g * L, L)]
            cvb = cv[slot][pl.ds(g * L, L)]

            def dbody(d, acc):
                mu = plsc.load_gather(gu[slot], [rows, cub + d])
                mv = plsc.load_gather(gv[slot], [rows, cvb + d])
                return acc + mu * mv

            acc = lax.fori_loop(0, D, dbody, jnp.zeros((L,), jnp.float32))
            e = jnp.exp(-acc)
            outb[pl.ds(ch * CH + g * L, L)] = e / (1.0 + e)
            return 0

        lax.fori_loop(0, CH // L, group, 0)

    handles = [None] * NBUF
    handles[0] = fire(0, 0)
    for ch in range(NCH):
        slot = ch % NBUF
        nxt = ch + 1
        if nxt < NCH:
            handles[nxt % NBUF] = fire(nxt, nxt % NBUF)
        for h in handles[slot]:
            h.wait()
        compute(ch, slot)

    pltpu.sync_copy(outb, out_hbm.at[pl.ds(base, BPW)])


def _score(u, v, mod):
    mesh = plsc.VectorSubcoreMesh(core_axis_name="c", subcore_axis_name="s")
    run = pl.kernel(
        _score_body,
        out_type=jax.ShapeDtypeStruct((BATCH,), jnp.float32),
        mesh=mesh,
        scratch_types=dict(
            uraw=[pltpu.VMEM((CH,), jnp.int32) for _ in range(NBUF)],
            vraw=[pltpu.VMEM((CH,), jnp.int32) for _ in range(NBUF)],
            pu=[pltpu.VMEM((CH,), jnp.int32) for _ in range(NBUF)],
            pv=[pltpu.VMEM((CH,), jnp.int32) for _ in range(NBUF)],
            cu=[pltpu.VMEM((CH,), jnp.int32) for _ in range(NBUF)],
            cv=[pltpu.VMEM((CH,), jnp.int32) for _ in range(NBUF)],
            gu=[pltpu.VMEM((CH, 2 * D), jnp.float32) for _ in range(NBUF)],
            gv=[pltpu.VMEM((CH, 2 * D), jnp.float32) for _ in range(NBUF)],
            outb=pltpu.VMEM((BPW,), jnp.float32),
            sems=[pltpu.SemaphoreType.DMA for _ in range(NBUF)],
        ),
        compiler_params=pltpu.CompilerParams(needs_layout_passes=False),
    )
    return run(u, v, mod)


@jax.jit
def _rotate_score(u, v, emb_re, emb_im):
    mod = _mod_table(emb_re.T, emb_im.T)
    return _score(u, v, mod)


def kernel(u, v, emb_re, emb_im):
    return _rotate_score(u.astype(jnp.int32), v.astype(jnp.int32),
                         emb_re, emb_im)
